# baseline (speedup 1.0000x reference)
import functools

import jax
import jax.numpy as jnp
from jax import lax
from jax.experimental import pallas as pl
from jax.experimental.pallas import tpu as pltpu
from jax.experimental.pallas import tpu_sc as plsc

T = 2048
D = 768
F = 2048
E = 8
EP = 128
TILE = 128
NTILES = T // TILE + E - 1
RS = NTILES * TILE
NC, NS = 2, 16
NW = NC * NS
RPW = T // NW

_F32 = jnp.float32
_I32 = jnp.int32


def _route_body(x_ref, wg_ref, slot_ref, te_ref):
    x = x_ref[...]
    wg = wg_ref[...]
    logits = jnp.dot(x, wg, preferred_element_type=_F32)
    lane = lax.broadcasted_iota(_I32, (T, EP), 1)
    logits = jnp.where(lane < E, logits, _F32(-1e30))
    m = jnp.max(logits, axis=1, keepdims=True)
    eid = jnp.min(jnp.where(logits >= m, lane, EP), axis=1, keepdims=True)
    oh = (lane == eid).astype(_F32)

    counts = jnp.sum(oh, axis=0, keepdims=True)
    nt = jnp.floor((counts + _F32(TILE - 1)) / _F32(TILE))
    r = lax.broadcasted_iota(_I32, (EP, EP), 0)
    c = lax.broadcasted_iota(_I32, (EP, EP), 1)
    incl = (r <= c).astype(_F32)
    ct = jnp.dot(nt, incl, preferred_element_type=_F32)
    p = _F32(TILE) * (ct - nt)

    rr = lax.broadcasted_iota(_I32, (T, T), 0)
    cc = lax.broadcasted_iota(_I32, (T, T), 1)
    tri = (cc <= rr).astype(_F32)
    cum = jnp.dot(tri, oh, preferred_element_type=_F32)
    slot = jnp.sum(oh * (cum - _F32(1.0) + p), axis=1, keepdims=True)
    slot_ref[...] = slot.astype(_I32)

    jrow = lax.broadcasted_iota(_I32, (EP, EP), 0).astype(_F32)
    owns = jnp.where(c < E, (jrow >= ct).astype(_F32), _F32(0.0))
    te = jnp.minimum(jnp.sum(owns, axis=1, keepdims=True), _F32(E - 1))
    te_ref[...] = te.astype(_I32)


_route = pl.pallas_call(
    _route_body,
    out_shape=[
        jax.ShapeDtypeStruct((T, 1), _I32),
        jax.ShapeDtypeStruct((EP, 1), _I32),
    ],
    compiler_params=pltpu.CompilerParams(
        vmem_limit_bytes=110 * 1024 * 1024),
)


_NBUF = 4
_LOOK = 8
_NCH = 4


def _ffn_body(te_ref, slot_ref, x_ref, wu_hbm, wd_hbm, ys_ref,
              wu_buf, wd_buf, flags, sem_u, sem_d):
    i = pl.program_id(0)
    n = pl.num_programs(0)
    e = te_ref[i]

    def cp(eq, slot):
        cu = [pltpu.make_async_copy(wu_hbm.at[eq, pl.ds(k * (D // _NCH), D // _NCH)],
                                    wu_buf.at[slot, pl.ds(k * (D // _NCH), D // _NCH)],
                                    sem_u.at[slot, k])
              for k in range(_NCH)]
        cd = [pltpu.make_async_copy(wd_hbm.at[eq, pl.ds(k * (F // _NCH), F // _NCH)],
                                    wd_buf.at[slot, pl.ds(k * (F // _NCH), F // _NCH)],
                                    sem_d.at[slot, k])
              for k in range(_NCH)]
        return cu + cd

    @pl.when(i == 0)
    def _init():
        for k in range(E):
            flags[k] = 0
        for c in cp(e, e % _NBUF):
            c.start()
        flags[e] = 1

    for k in range(1, _LOOK + 1):
        idx = jnp.minimum(i + k, n - 1)
        ek = te_ref[idx]

        @pl.when((flags[ek] == 0) & (ek < e + _NBUF))
        def _prefetch(ek=ek):
            for c in cp(ek, ek % _NBUF):
                c.start()
            flags[ek] = 1

    @pl.when(flags[e] == 1)
    def _wait():
        for c in cp(e, e % _NBUF):
            c.wait()
        flags[e] = 2

    slot = e % _NBUF
    lane = lax.broadcasted_iota(_I32, (T, TILE), 1)
    q = (slot_ref[...] == i * TILE + lane).astype(_F32)
    xs = lax.dot_general(q, x_ref[...], (((0,), (0,)), ((), ())),
                         preferred_element_type=_F32)
    h = jnp.dot(xs, wu_buf[slot], preferred_element_type=_F32)
    h = jax.nn.gelu(h)
    ys_ref[...] = jnp.dot(h, wd_buf[slot], preferred_element_type=_F32)


_ffn = pl.pallas_call(
    _ffn_body,
    grid_spec=pltpu.PrefetchScalarGridSpec(
        num_scalar_prefetch=1,
        grid=(NTILES,),
        in_specs=[
            pl.BlockSpec((T, 1), lambda i, te: (0, 0)),
            pl.BlockSpec((T, D), lambda i, te: (0, 0)),
            pl.BlockSpec(memory_space=pltpu.MemorySpace.HBM),
            pl.BlockSpec(memory_space=pltpu.MemorySpace.HBM),
        ],
        out_specs=pl.BlockSpec((TILE, D), lambda i, te: (i, 0)),
        scratch_shapes=[
            pltpu.VMEM((_NBUF, D, F), _F32),
            pltpu.VMEM((_NBUF, F, D), _F32),
            pltpu.SMEM((E,), _I32),
            pltpu.SemaphoreType.DMA((_NBUF, _NCH)),
            pltpu.SemaphoreType.DMA((_NBUF, _NCH)),
        ],
    ),
    out_shape=jax.ShapeDtypeStruct((RS, D), _F32),
    compiler_params=pltpu.CompilerParams(
        dimension_semantics=("arbitrary",),
        vmem_limit_bytes=110 * 1024 * 1024,
    ),
)


def _sc_mesh():
    return plsc.VectorSubcoreMesh(
        core_axis_name="c", subcore_axis_name="s",
        num_cores=NC, num_subcores=NS)


def _make_combine():
    @functools.partial(
        pl.kernel,
        out_type=jax.ShapeDtypeStruct((T, D), _F32),
        mesh=_sc_mesh(),
        scratch_types=[
            pltpu.VMEM((RPW,), _I32),
            pltpu.VMEM((RPW, D), _F32),
            pltpu.SemaphoreType.DMA,
        ],
    )
    def combine(ys_hbm, slot_hbm, out_hbm, idx_v, rows_v, sem):
        wid = lax.axis_index("s") * NC + lax.axis_index("c")
        base = wid * RPW
        pltpu.sync_copy(slot_hbm.at[pl.ds(base, RPW)], idx_v)
        pltpu.async_copy(ys_hbm.at[idx_v], rows_v, sem).wait()
        pltpu.sync_copy(rows_v, out_hbm.at[pl.ds(base, RPW)])

    return combine


def kernel(x, Wg, W_up, W_down):
    B, S, d = x.shape
    t = x.reshape(B * S, d)
    wgp = jnp.pad(Wg, ((0, 0), (0, EP - E)))
    slot_col, te_col = _route(t, wgp)
    slot = slot_col.reshape(B * S)
    te = te_col.reshape(EP)[:NTILES]
    ys = _ffn(te, slot_col, t, W_up, W_down)
    out = _make_combine()(ys, slot)
    return out.reshape(B, S, d)


# speedup vs baseline: 1.0762x; 1.0762x over previous
import functools

import jax
import jax.numpy as jnp
from jax import lax
from jax.experimental import pallas as pl
from jax.experimental.pallas import tpu as pltpu
from jax.experimental.pallas import tpu_sc as plsc

T = 2048
D = 768
F = 2048
E = 8
EP = 128
TILE = 128
NTILES = T // TILE + E - 1
RS = NTILES * TILE
NC, NS = 2, 16
NW = NC * NS
RPW = T // NW

_F32 = jnp.float32
_I32 = jnp.int32


def _route_body(x_ref, wg_ref, slot_ref, srow_ref, te_ref):
    x = x_ref[...]
    wg = wg_ref[...]
    logits = jnp.dot(x, wg, preferred_element_type=_F32)
    lane = lax.broadcasted_iota(_I32, (T, EP), 1)
    logits = jnp.where(lane < E, logits, _F32(-1e30))
    m = jnp.max(logits, axis=1, keepdims=True)
    eid = jnp.min(jnp.where(logits >= m, lane, EP), axis=1, keepdims=True)
    oh = (lane == eid).astype(_F32)

    counts = jnp.sum(oh, axis=0, keepdims=True)
    nt = jnp.floor((counts + _F32(TILE - 1)) / _F32(TILE))
    r = lax.broadcasted_iota(_I32, (EP, EP), 0)
    c = lax.broadcasted_iota(_I32, (EP, EP), 1)
    incl = (r <= c).astype(_F32)
    ct = jnp.dot(nt, incl, preferred_element_type=_F32)
    p = _F32(TILE) * (ct - nt)

    rr = lax.broadcasted_iota(_I32, (T, T), 0)
    cc = lax.broadcasted_iota(_I32, (T, T), 1)
    tri = (cc <= rr).astype(_F32)
    cum = jnp.dot(tri, oh, preferred_element_type=_F32)
    slot = jnp.sum(oh * (cum - _F32(1.0) + p), axis=1, keepdims=True)
    slot_ref[...] = slot.astype(_I32)
    srow_ref[...] = slot.astype(_I32).reshape(1, T)

    jrow = lax.broadcasted_iota(_I32, (EP, EP), 0).astype(_F32)
    owns = jnp.where(c < E, (jrow >= ct).astype(_F32), _F32(0.0))
    te = jnp.minimum(jnp.sum(owns, axis=1, keepdims=True), _F32(E - 1))
    te_ref[...] = te.astype(_I32)


_route = pl.pallas_call(
    _route_body,
    out_shape=[
        jax.ShapeDtypeStruct((T, 1), _I32),
        jax.ShapeDtypeStruct((1, T), _I32),
        jax.ShapeDtypeStruct((EP, 1), _I32),
    ],
    compiler_params=pltpu.CompilerParams(
        vmem_limit_bytes=110 * 1024 * 1024),
)


_NBUF = 4
_LOOK = 8
_NCH = 4


def _ffn_body(te_ref, srow_ref, x_ref, wu_hbm, wd_hbm, ys_ref,
              wu_buf, wd_buf, flags, sem_u, sem_d):
    i = pl.program_id(0)
    n = pl.num_programs(0)
    e = te_ref[i]

    def cp(eq, slot):
        cu = [pltpu.make_async_copy(wu_hbm.at[eq, pl.ds(k * (D // _NCH), D // _NCH)],
                                    wu_buf.at[slot, pl.ds(k * (D // _NCH), D // _NCH)],
                                    sem_u.at[slot, k])
              for k in range(_NCH)]
        cd = [pltpu.make_async_copy(wd_hbm.at[eq, pl.ds(k * (F // _NCH), F // _NCH)],
                                    wd_buf.at[slot, pl.ds(k * (F // _NCH), F // _NCH)],
                                    sem_d.at[slot, k])
              for k in range(_NCH)]
        return cu + cd

    @pl.when(i == 0)
    def _init():
        for k in range(E):
            flags[k] = 0
        for c in cp(e, e % _NBUF):
            c.start()
        flags[e] = 1

    for k in range(1, _LOOK + 1):
        idx = jnp.minimum(i + k, n - 1)
        ek = te_ref[idx]

        @pl.when((flags[ek] == 0) & (ek < e + _NBUF))
        def _prefetch(ek=ek):
            for c in cp(ek, ek % _NBUF):
                c.start()
            flags[ek] = 1

    @pl.when(flags[e] == 1)
    def _wait():
        for c in cp(e, e % _NBUF):
            c.wait()
        flags[e] = 2

    slot = e % _NBUF
    r = lax.broadcasted_iota(_I32, (TILE, T), 0)
    qt = (srow_ref[...] == i * TILE + r).astype(_F32)
    xs = jnp.dot(qt, x_ref[...], preferred_element_type=_F32)
    h = jnp.dot(xs, wu_buf[slot], preferred_element_type=_F32)
    h = jax.nn.gelu(h)
    ys_ref[...] = jnp.dot(h, wd_buf[slot], preferred_element_type=_F32)


_ffn = pl.pallas_call(
    _ffn_body,
    grid_spec=pltpu.PrefetchScalarGridSpec(
        num_scalar_prefetch=1,
        grid=(NTILES,),
        in_specs=[
            pl.BlockSpec((1, T), lambda i, te: (0, 0)),
            pl.BlockSpec((T, D), lambda i, te: (0, 0)),
            pl.BlockSpec(memory_space=pltpu.MemorySpace.HBM),
            pl.BlockSpec(memory_space=pltpu.MemorySpace.HBM),
        ],
        out_specs=pl.BlockSpec((TILE, D), lambda i, te: (i, 0)),
        scratch_shapes=[
            pltpu.VMEM((_NBUF, D, F), _F32),
            pltpu.VMEM((_NBUF, F, D), _F32),
            pltpu.SMEM((E,), _I32),
            pltpu.SemaphoreType.DMA((_NBUF, _NCH)),
            pltpu.SemaphoreType.DMA((_NBUF, _NCH)),
        ],
    ),
    out_shape=jax.ShapeDtypeStruct((RS, D), _F32),
    compiler_params=pltpu.CompilerParams(
        dimension_semantics=("arbitrary",),
        vmem_limit_bytes=110 * 1024 * 1024,
    ),
)


def _sc_mesh():
    return plsc.VectorSubcoreMesh(
        core_axis_name="c", subcore_axis_name="s",
        num_cores=NC, num_subcores=NS)


def _make_combine():
    @functools.partial(
        pl.kernel,
        out_type=jax.ShapeDtypeStruct((T, D), _F32),
        mesh=_sc_mesh(),
        scratch_types=[
            pltpu.VMEM((RPW,), _I32),
            pltpu.VMEM((RPW, D), _F32),
            pltpu.SemaphoreType.DMA,
        ],
    )
    def combine(ys_hbm, slot_hbm, out_hbm, idx_v, rows_v, sem):
        wid = lax.axis_index("s") * NC + lax.axis_index("c")
        base = wid * RPW
        pltpu.sync_copy(slot_hbm.at[pl.ds(base, RPW)], idx_v)
        pltpu.async_copy(ys_hbm.at[idx_v], rows_v, sem).wait()
        pltpu.sync_copy(rows_v, out_hbm.at[pl.ds(base, RPW)])

    return combine


def kernel(x, Wg, W_up, W_down):
    B, S, d = x.shape
    t = x.reshape(B * S, d)
    wgp = jnp.pad(Wg, ((0, 0), (0, EP - E)))
    slot_col, slot_row, te_col = _route(t, wgp)
    slot = slot_col.reshape(B * S)
    te = te_col.reshape(EP)[:NTILES]
    ys = _ffn(te, slot_row, t, W_up, W_down)
    out = _make_combine()(ys, slot)
    return out.reshape(B, S, d)


# speedup vs baseline: 1.0829x; 1.0062x over previous
import functools

import jax
import jax.numpy as jnp
from jax import lax
from jax.experimental import pallas as pl
from jax.experimental.pallas import tpu as pltpu
from jax.experimental.pallas import tpu_sc as plsc

T = 2048
D = 768
F = 2048
E = 8
EP = 128
TILE = 128
NTILES = T // TILE + E - 1
RS = NTILES * TILE
NC, NS = 2, 16
NW = NC * NS
RPW = T // NW

_F32 = jnp.float32
_I32 = jnp.int32


def _route_body(x_ref, wg_ref, slot_ref, srow_ref, te_ref):
    x = x_ref[...]
    wg = wg_ref[...]
    logits = jnp.dot(x, wg, preferred_element_type=_F32)
    lane = lax.broadcasted_iota(_I32, (T, EP), 1)
    logits = jnp.where(lane < E, logits, _F32(-1e30))
    m = jnp.max(logits, axis=1, keepdims=True)
    eid = jnp.min(jnp.where(logits >= m, lane, EP), axis=1, keepdims=True)
    oh = (lane == eid).astype(_F32)

    counts = jnp.sum(oh, axis=0, keepdims=True)
    nt = jnp.floor((counts + _F32(TILE - 1)) / _F32(TILE))
    r = lax.broadcasted_iota(_I32, (EP, EP), 0)
    c = lax.broadcasted_iota(_I32, (EP, EP), 1)
    incl = (r <= c).astype(_F32)
    ct = jnp.dot(nt, incl, preferred_element_type=_F32)
    p = _F32(TILE) * (ct - nt)

    rr = lax.broadcasted_iota(_I32, (T, T), 0)
    cc = lax.broadcasted_iota(_I32, (T, T), 1)
    tri = (cc <= rr).astype(_F32)
    cum = jnp.dot(tri, oh, preferred_element_type=_F32)
    slot = jnp.sum(oh * (cum - _F32(1.0) + p), axis=1, keepdims=True)
    slot_ref[...] = slot.astype(_I32)
    srow_ref[...] = slot.astype(_I32).reshape(1, T)

    jrow = lax.broadcasted_iota(_I32, (EP, EP), 0).astype(_F32)
    owns = jnp.where(c < E, (jrow >= ct).astype(_F32), _F32(0.0))
    te = jnp.minimum(jnp.sum(owns, axis=1, keepdims=True), _F32(E - 1))
    te_ref[...] = te.astype(_I32)


_route = pl.pallas_call(
    _route_body,
    out_shape=[
        jax.ShapeDtypeStruct((T, 1), _I32),
        jax.ShapeDtypeStruct((1, T), _I32),
        jax.ShapeDtypeStruct((EP, 1), _I32),
    ],
    compiler_params=pltpu.CompilerParams(
        vmem_limit_bytes=110 * 1024 * 1024),
)


_NBUF = 4
_LOOK = 8
_NCH = 4


_BF16 = jnp.bfloat16


def _ffn_body(te_ref, srow_ref, x_ref, wu_hbm, wd_hbm, ys_ref,
              wu_buf, wd_buf, xb, flags, sem_u, sem_d):
    i = pl.program_id(0)
    n = pl.num_programs(0)
    e = te_ref[i]

    def cp(eq, slot):
        cu = [pltpu.make_async_copy(wu_hbm.at[eq, pl.ds(k * (D // _NCH), D // _NCH)],
                                    wu_buf.at[slot, pl.ds(k * (D // _NCH), D // _NCH)],
                                    sem_u.at[slot, k])
              for k in range(_NCH)]
        cd = [pltpu.make_async_copy(wd_hbm.at[eq, pl.ds(k * (F // _NCH), F // _NCH)],
                                    wd_buf.at[slot, pl.ds(k * (F // _NCH), F // _NCH)],
                                    sem_d.at[slot, k])
              for k in range(_NCH)]
        return cu + cd

    @pl.when(i == 0)
    def _init():
        for k in range(E):
            flags[k] = 0
        for c in cp(e, e % _NBUF):
            c.start()
        flags[e] = 1
        xb[...] = x_ref[...].astype(_BF16)

    for k in range(1, _LOOK + 1):
        idx = jnp.minimum(i + k, n - 1)
        ek = te_ref[idx]

        @pl.when((flags[ek] == 0) & (ek < e + _NBUF))
        def _prefetch(ek=ek):
            for c in cp(ek, ek % _NBUF):
                c.start()
            flags[ek] = 1

    slot = e % _NBUF

    @pl.when(flags[e] == 1)
    def _wait():
        for c in cp(e, e % _NBUF):
            c.wait()
        flags[e] = 2

    r = lax.broadcasted_iota(_I32, (TILE, T), 0)
    qt = (srow_ref[...] == i * TILE + r).astype(_BF16)
    xs = jnp.dot(qt, xb[...], preferred_element_type=_F32)
    h = jnp.dot(xs, wu_buf[slot], preferred_element_type=_F32)
    h = jax.nn.gelu(h)
    ys_ref[...] = jnp.dot(h, wd_buf[slot], preferred_element_type=_F32)


_ffn = pl.pallas_call(
    _ffn_body,
    grid_spec=pltpu.PrefetchScalarGridSpec(
        num_scalar_prefetch=1,
        grid=(NTILES,),
        in_specs=[
            pl.BlockSpec((1, T), lambda i, te: (0, 0)),
            pl.BlockSpec((T, D), lambda i, te: (0, 0)),
            pl.BlockSpec(memory_space=pltpu.MemorySpace.HBM),
            pl.BlockSpec(memory_space=pltpu.MemorySpace.HBM),
        ],
        out_specs=pl.BlockSpec((TILE, D), lambda i, te: (i, 0)),
        scratch_shapes=[
            pltpu.VMEM((_NBUF, D, F), _F32),
            pltpu.VMEM((_NBUF, F, D), _F32),
            pltpu.VMEM((T, D), _BF16),
            pltpu.SMEM((E,), _I32),
            pltpu.SemaphoreType.DMA((_NBUF, _NCH)),
            pltpu.SemaphoreType.DMA((_NBUF, _NCH)),
        ],
    ),
    out_shape=jax.ShapeDtypeStruct((RS, D), _F32),
    compiler_params=pltpu.CompilerParams(
        dimension_semantics=("arbitrary",),
        vmem_limit_bytes=110 * 1024 * 1024,
    ),
)


def _sc_mesh():
    return plsc.VectorSubcoreMesh(
        core_axis_name="c", subcore_axis_name="s",
        num_cores=NC, num_subcores=NS)


def _make_combine():
    @functools.partial(
        pl.kernel,
        out_type=jax.ShapeDtypeStruct((T, D), _F32),
        mesh=_sc_mesh(),
        scratch_types=[
            pltpu.VMEM((RPW,), _I32),
            pltpu.VMEM((RPW, D), _F32),
            pltpu.SemaphoreType.DMA,
        ],
    )
    def combine(ys_hbm, slot_hbm, out_hbm, idx_v, rows_v, sem):
        wid = lax.axis_index("s") * NC + lax.axis_index("c")
        base = wid * RPW
        pltpu.sync_copy(slot_hbm.at[pl.ds(base, RPW)], idx_v)
        pltpu.async_copy(ys_hbm.at[idx_v], rows_v, sem).wait()
        pltpu.sync_copy(rows_v, out_hbm.at[pl.ds(base, RPW)])

    return combine


def kernel(x, Wg, W_up, W_down):
    B, S, d = x.shape
    t = x.reshape(B * S, d)
    wgp = jnp.pad(Wg, ((0, 0), (0, EP - E)))
    slot_col, slot_row, te_col = _route(t, wgp)
    slot = slot_col.reshape(B * S)
    te = te_col.reshape(EP)[:NTILES]
    ys = _ffn(te, slot_row, t, W_up, W_down)
    out = _make_combine()(ys, slot)
    return out.reshape(B, S, d)
